# baseline (speedup 1.0000x reference)
import functools

import jax
import jax.numpy as jnp
from jax.experimental import pallas as pl
from jax.experimental.pallas import tpu as pltpu
from jax.experimental.pallas import tpu_sc as plsc

_FEATURES = 192
_FEATURES_PAD = 256
_CODE_FEATURES = 32
_PAGES = 1024
_N_TOKENS = 2048
_BETA = 0.25
_EPS = 1e-12
_TOK_BLK = 512
_N_HEAD = 512
_N_TAIL = _N_TOKENS - _N_HEAD
_NBLK_TAIL = _N_TAIL // _TOK_BLK
_LOSS_SCALE = (1.0 + _BETA) / (_N_TOKENS * _CODE_FEATURES)


def _codebook_prep(cb_ref):
    cb = cb_ref[...]
    cn = cb * jax.lax.rsqrt(
        jnp.mean(cb * cb, axis=1, keepdims=True) + _EPS)
    cnT = cn.T
    c2 = jnp.sum(cnT * cnT, axis=0, keepdims=True)
    return cn, cnT, c2


def _search(z, w_in_ref, b_in_ref, cnT, c2):
    zp = jnp.dot(z, w_in_ref[...],
                 preferred_element_type=jnp.float32) + b_in_ref[...]
    zn = zp * jax.lax.rsqrt(
        jnp.mean(zp * zp, axis=1, keepdims=True) + _EPS)
    dots = jnp.dot(zn, cnT, preferred_element_type=jnp.float32,
                   precision=jax.lax.Precision.HIGHEST)
    dist = c2 - 2.0 * dots
    m = jnp.min(dist, axis=1, keepdims=True)
    lane = jax.lax.broadcasted_iota(jnp.int32, dist.shape, 1)
    idx = jnp.min(jnp.where(dist == m, lane, _PAGES), axis=1)
    return zn, m, idx


def _tc_head_body(z_ref, cb_ref, w_in_ref, b_in_ref, w_out_ref, b_out_ref,
                  idx_ref, loss_ref, p_ref):
    cn, cnT, c2 = _codebook_prep(cb_ref)
    p = jnp.dot(cn, w_out_ref[...],
                preferred_element_type=jnp.float32) + b_out_ref[...]
    p_ref[...] = jnp.pad(p, ((0, 0), (0, _FEATURES_PAD - _FEATURES)))
    zn, m, idx = _search(z_ref[...], w_in_ref, b_in_ref, cnT, c2)
    idx_ref[0, 0, :] = idx
    loss_ref[...] = jnp.full((1, 1), jnp.sum(zn * zn) + jnp.sum(m),
                             jnp.float32)


_tc_head = pl.pallas_call(
    _tc_head_body,
    grid=(1,),
    in_specs=[
        pl.BlockSpec((_N_HEAD, _FEATURES), lambda i: (0, 0)),
        pl.BlockSpec((_PAGES, _CODE_FEATURES), lambda i: (0, 0)),
        pl.BlockSpec((_FEATURES, _CODE_FEATURES), lambda i: (0, 0)),
        pl.BlockSpec((1, _CODE_FEATURES), lambda i: (0, 0)),
        pl.BlockSpec((_CODE_FEATURES, _FEATURES), lambda i: (0, 0)),
        pl.BlockSpec((1, _FEATURES), lambda i: (0, 0)),
    ],
    out_specs=[
        pl.BlockSpec((1, 1, _N_HEAD), lambda i: (0, 0, 0)),
        pl.BlockSpec((1, 1), lambda i: (0, 0)),
        pl.BlockSpec((_PAGES, _FEATURES_PAD), lambda i: (0, 0)),
    ],
    out_shape=[
        jax.ShapeDtypeStruct((1, 1, _N_HEAD), jnp.int32),
        jax.ShapeDtypeStruct((1, 1), jnp.float32),
        jax.ShapeDtypeStruct((_PAGES, _FEATURES_PAD), jnp.float32),
    ],
)


def _tc_tail_body(z_ref, cb_ref, w_in_ref, b_in_ref, lossa_ref,
                  idx_ref, loss_ref, cnT_s, c2_s):
    step = pl.program_id(0)

    @pl.when(step == 0)
    def _prep():
        _, cnT, c2 = _codebook_prep(cb_ref)
        cnT_s[...] = cnT
        c2_s[...] = c2
        loss_ref[...] = lossa_ref[...]

    zn, m, idx = _search(z_ref[...], w_in_ref, b_in_ref, cnT_s[...], c2_s[...])
    idx_ref[0, 0, :] = idx
    loss_ref[...] += jnp.full((1, 1), jnp.sum(zn * zn) + jnp.sum(m),
                              jnp.float32)

    @pl.when(step == _NBLK_TAIL - 1)
    def _finish():
        loss_ref[...] *= _LOSS_SCALE


_tc_tail = pl.pallas_call(
    _tc_tail_body,
    grid=(_NBLK_TAIL,),
    in_specs=[
        pl.BlockSpec((_TOK_BLK, _FEATURES), lambda i: (i, 0)),
        pl.BlockSpec((_PAGES, _CODE_FEATURES), lambda i: (0, 0)),
        pl.BlockSpec((_FEATURES, _CODE_FEATURES), lambda i: (0, 0)),
        pl.BlockSpec((1, _CODE_FEATURES), lambda i: (0, 0)),
        pl.BlockSpec((1, 1), lambda i: (0, 0)),
    ],
    out_specs=[
        pl.BlockSpec((1, 1, _TOK_BLK), lambda i: (i, 0, 0)),
        pl.BlockSpec((1, 1), lambda i: (0, 0)),
    ],
    out_shape=[
        jax.ShapeDtypeStruct((_NBLK_TAIL, 1, _TOK_BLK), jnp.int32),
        jax.ShapeDtypeStruct((1, 1), jnp.float32),
    ],
    scratch_shapes=[
        pltpu.VMEM((_CODE_FEATURES, _PAGES), jnp.float32),
        pltpu.VMEM((1, _PAGES), jnp.float32),
    ],
)


def _sc_head_body(nc, bpw, idx_hbm, table_hbm, out_hbm, idx_v, rows_v, sem):
    wid = jax.lax.axis_index("s") * nc + jax.lax.axis_index("c")
    base = wid * bpw
    pltpu.sync_copy(idx_hbm.at[pl.ds(base, bpw)], idx_v)
    pltpu.async_copy(table_hbm.at[idx_v], rows_v, sem).wait()
    pltpu.sync_copy(rows_v, out_hbm.at[pl.ds(base, bpw)])


def _sc_tail_body(nc, bpw, hpw, idx_hbm, table_hbm, head_hbm, out_hbm,
                  idx_v, rows_v, sem):
    wid = jax.lax.axis_index("s") * nc + jax.lax.axis_index("c")
    hbase = wid * hpw
    pltpu.sync_copy(head_hbm.at[pl.ds(hbase, hpw)],
                    out_hbm.at[pl.ds(hbase, hpw)])
    base = wid * bpw
    pltpu.sync_copy(idx_hbm.at[pl.ds(base, bpw)], idx_v)
    pltpu.async_copy(table_hbm.at[idx_v], rows_v, sem).wait()
    pltpu.sync_copy(rows_v, out_hbm.at[pl.ds(_N_HEAD + base, bpw)])


def _sc_calls():
    info = plsc.get_sparse_core_info()
    nc, ns = info.num_cores, info.num_subcores
    nw = nc * ns
    mesh = plsc.VectorSubcoreMesh(core_axis_name="c", subcore_axis_name="s")
    hpw = _N_HEAD // nw
    tpw = _N_TAIL // nw
    head = pl.kernel(
        functools.partial(_sc_head_body, nc, hpw),
        out_type=jax.ShapeDtypeStruct((_N_HEAD, _FEATURES_PAD), jnp.float32),
        mesh=mesh,
        scratch_types=[
            pltpu.VMEM((hpw,), jnp.int32),
            pltpu.VMEM((hpw, _FEATURES_PAD), jnp.float32),
            pltpu.SemaphoreType.DMA,
        ],
    )
    tail = pl.kernel(
        functools.partial(_sc_tail_body, nc, tpw, hpw),
        out_type=jax.ShapeDtypeStruct((_N_TOKENS, _FEATURES_PAD), jnp.float32),
        mesh=mesh,
        scratch_types=[
            pltpu.VMEM((tpw,), jnp.int32),
            pltpu.VMEM((tpw, _FEATURES_PAD), jnp.float32),
            pltpu.SemaphoreType.DMA,
        ],
    )
    return head, tail


def kernel(z, codebook, W_in, b_in, W_out, b_out):
    b_in2 = b_in.reshape(1, -1)
    sc_head, sc_tail = _sc_calls()
    idxa3, lossa, table = _tc_head(
        z[:_N_HEAD], codebook, W_in, b_in2, W_out, b_out.reshape(1, -1))
    idxa = idxa3.reshape(_N_HEAD)
    out_head = sc_head(idxa, table)
    idxb3, loss11 = _tc_tail(z[_N_HEAD:], codebook, W_in, b_in2, lossa)
    idxb = idxb3.reshape(_N_TAIL)
    out = sc_tail(idxb, table, out_head)
    idxes = jnp.concatenate([idxa, idxb])
    return (out[:, :_FEATURES], loss11.reshape(()), idxes)


# speedup vs baseline: 1.4510x; 1.4510x over previous
import functools

import jax
import jax.numpy as jnp
from jax.experimental import pallas as pl
from jax.experimental.pallas import tpu as pltpu
from jax.experimental.pallas import tpu_sc as plsc

_FEATURES = 192
_FEATURES_PAD = 256
_CODE_FEATURES = 32
_PAGES = 1024
_N_TOKENS = 2048
_BETA = 0.25
_EPS = 1e-12
_TOK_BLK = 512
_NBLK = _N_TOKENS // _TOK_BLK


def _tc_body(z_ref, cb_ref, w_in_ref, b_in_ref, w_out_ref,
             b_out_ref, idx_ref, loss_ref, p_ref, b_s):
    step = pl.program_id(0)

    @pl.when(step == 0)
    def _prep():
        cb = cb_ref[...]
        cn = cb * jax.lax.rsqrt(
            jnp.mean(cb * cb, axis=1, keepdims=True) + _EPS)
        cnT = cn.T
        c2 = jnp.sum(cnT * cnT, axis=0, keepdims=True)
        b_s[...] = jnp.concatenate([cnT * -2.0, c2], axis=0)
        p = jnp.dot(cn, w_out_ref[...],
                    preferred_element_type=jnp.float32) + b_out_ref[...]
        p_ref[...] = jnp.pad(p, ((0, 0), (0, _FEATURES_PAD - _FEATURES)))

    z = z_ref[...]
    zp = jnp.dot(z, w_in_ref[...],
                 preferred_element_type=jnp.float32) + b_in_ref[...]
    zn = zp * jax.lax.rsqrt(
        jnp.mean(zp * zp, axis=1, keepdims=True) + _EPS)

    zn_aug = jnp.concatenate(
        [zn, jnp.ones((_TOK_BLK, 1), jnp.float32)], axis=1)
    dist = jnp.dot(zn_aug, b_s[...], preferred_element_type=jnp.float32,
                   precision=jax.lax.Precision.HIGHEST)
    m = jnp.min(dist, axis=1, keepdims=True)
    lane = jax.lax.broadcasted_iota(jnp.int32, dist.shape, 1)
    idx = jnp.min(jnp.where(dist == m, lane, _PAGES), axis=1)
    idx_ref[0, 0, :] = idx

    @pl.when(step == 0)
    def _init():
        loss_ref[...] = jnp.zeros_like(loss_ref)

    partial = jnp.sum(zn * zn) + jnp.sum(m)
    loss_ref[...] += jnp.full((1, 1), partial, jnp.float32)

    @pl.when(step == _NBLK - 1)
    def _finish():
        loss_ref[...] *= (1.0 + _BETA) / (_N_TOKENS * _CODE_FEATURES)


_tc_call = pl.pallas_call(
    _tc_body,
    grid=(_NBLK,),
    in_specs=[
        pl.BlockSpec((_TOK_BLK, _FEATURES), lambda i: (i, 0)),
        pl.BlockSpec((_PAGES, _CODE_FEATURES), lambda i: (0, 0)),
        pl.BlockSpec((_FEATURES, _CODE_FEATURES), lambda i: (0, 0)),
        pl.BlockSpec((1, _CODE_FEATURES), lambda i: (0, 0)),
        pl.BlockSpec((_CODE_FEATURES, _FEATURES), lambda i: (0, 0)),
        pl.BlockSpec((1, _FEATURES), lambda i: (0, 0)),
    ],
    out_specs=[
        pl.BlockSpec((1, 1, _TOK_BLK), lambda i: (i, 0, 0)),
        pl.BlockSpec((1, 1), lambda i: (0, 0)),
        pl.BlockSpec((_PAGES, _FEATURES_PAD), lambda i: (0, 0)),
    ],
    out_shape=[
        jax.ShapeDtypeStruct((_NBLK, 1, _TOK_BLK), jnp.int32),
        jax.ShapeDtypeStruct((1, 1), jnp.float32),
        jax.ShapeDtypeStruct((_PAGES, _FEATURES_PAD), jnp.float32),
    ],
    scratch_shapes=[
        pltpu.VMEM((_CODE_FEATURES + 1, _PAGES), jnp.float32),
    ],
)


def _sc_gather_body(nc, bpw, idx_hbm, table_hbm, out_hbm, idx_v, rows_v, sem):
    wid = jax.lax.axis_index("s") * nc + jax.lax.axis_index("c")
    base = wid * bpw
    pltpu.sync_copy(idx_hbm.at[pl.ds(base, bpw)], idx_v)
    pltpu.async_copy(table_hbm.at[idx_v], rows_v, sem).wait()
    pltpu.sync_copy(rows_v, out_hbm.at[pl.ds(base, bpw)])


def _sc_gather(idxes, table):
    info = plsc.get_sparse_core_info()
    nc, ns = info.num_cores, info.num_subcores
    bpw = _N_TOKENS // (nc * ns)
    call = pl.kernel(
        functools.partial(_sc_gather_body, nc, bpw),
        out_type=jax.ShapeDtypeStruct((_N_TOKENS, _FEATURES_PAD), jnp.float32),
        mesh=plsc.VectorSubcoreMesh(core_axis_name="c", subcore_axis_name="s"),
        scratch_types=[
            pltpu.VMEM((bpw,), jnp.int32),
            pltpu.VMEM((bpw, _FEATURES_PAD), jnp.float32),
            pltpu.SemaphoreType.DMA,
        ],
    )
    return call(idxes, table)


def kernel(z, codebook, W_in, b_in, W_out, b_out):
    idx3, loss11, table = _tc_call(
        z, codebook, W_in, b_in.reshape(1, -1), W_out, b_out.reshape(1, -1))
    idxes = idx3.reshape(_N_TOKENS)
    out = _sc_gather(idxes, table)
    return (out[:, :_FEATURES], loss11.reshape(()), idxes)


# speedup vs baseline: 1.4830x; 1.0221x over previous
import functools

import jax
import jax.numpy as jnp
from jax.experimental import pallas as pl
from jax.experimental.pallas import tpu as pltpu
from jax.experimental.pallas import tpu_sc as plsc

_FEATURES = 192
_FEATURES_PAD = 256
_CODE_FEATURES = 32
_PAGES = 1024
_N_TOKENS = 2048
_BETA = 0.25
_EPS = 1e-12
_TOK_BLK = 1024
_NBLK = _N_TOKENS // _TOK_BLK


def _tc_body(z_ref, cb_ref, w_in_ref, b_in_ref, w_out_ref,
             b_out_ref, idx_ref, loss_ref, p_ref, b_s):
    step = pl.program_id(0)

    @pl.when(step == 0)
    def _prep():
        cb = cb_ref[...]
        cn = cb * jax.lax.rsqrt(
            jnp.mean(cb * cb, axis=1, keepdims=True) + _EPS)
        cnT = cn.T
        c2 = jnp.sum(cnT * cnT, axis=0, keepdims=True)
        b_s[...] = jnp.concatenate([cnT * -2.0, c2], axis=0)
        p = jnp.dot(cn, w_out_ref[...],
                    preferred_element_type=jnp.float32) + b_out_ref[...]
        p_ref[...] = jnp.pad(p, ((0, 0), (0, _FEATURES_PAD - _FEATURES)))

    z = z_ref[...]
    zp = jnp.dot(z, w_in_ref[...],
                 preferred_element_type=jnp.float32) + b_in_ref[...]
    zn = zp * jax.lax.rsqrt(
        jnp.mean(zp * zp, axis=1, keepdims=True) + _EPS)

    zn_aug = jnp.concatenate(
        [zn, jnp.ones((_TOK_BLK, 1), jnp.float32)], axis=1)
    dist = jnp.dot(zn_aug, b_s[...], preferred_element_type=jnp.float32,
                   precision=jax.lax.Precision.HIGHEST)
    m = jnp.min(dist, axis=1, keepdims=True)
    lane = jax.lax.broadcasted_iota(jnp.int32, dist.shape, 1)
    idx = jnp.min(jnp.where(dist == m, lane, _PAGES), axis=1)
    idx_ref[0, 0, :] = idx

    @pl.when(step == 0)
    def _init():
        loss_ref[...] = jnp.zeros_like(loss_ref)

    partial = jnp.sum(zn * zn) + jnp.sum(m)
    loss_ref[...] += jnp.full((1, 1), partial, jnp.float32)

    @pl.when(step == _NBLK - 1)
    def _finish():
        loss_ref[...] *= (1.0 + _BETA) / (_N_TOKENS * _CODE_FEATURES)


_tc_call = pl.pallas_call(
    _tc_body,
    grid=(_NBLK,),
    in_specs=[
        pl.BlockSpec((_TOK_BLK, _FEATURES), lambda i: (i, 0)),
        pl.BlockSpec((_PAGES, _CODE_FEATURES), lambda i: (0, 0)),
        pl.BlockSpec((_FEATURES, _CODE_FEATURES), lambda i: (0, 0)),
        pl.BlockSpec((1, _CODE_FEATURES), lambda i: (0, 0)),
        pl.BlockSpec((_CODE_FEATURES, _FEATURES), lambda i: (0, 0)),
        pl.BlockSpec((1, _FEATURES), lambda i: (0, 0)),
    ],
    out_specs=[
        pl.BlockSpec((1, 1, _TOK_BLK), lambda i: (i, 0, 0)),
        pl.BlockSpec((1, 1), lambda i: (0, 0)),
        pl.BlockSpec((_PAGES, _FEATURES_PAD), lambda i: (0, 0)),
    ],
    out_shape=[
        jax.ShapeDtypeStruct((_NBLK, 1, _TOK_BLK), jnp.int32),
        jax.ShapeDtypeStruct((1, 1), jnp.float32),
        jax.ShapeDtypeStruct((_PAGES, _FEATURES_PAD), jnp.float32),
    ],
    scratch_shapes=[
        pltpu.VMEM((_CODE_FEATURES + 1, _PAGES), jnp.float32),
    ],
)


def _sc_gather_body(nc, bpw, idx_hbm, table_hbm, out_hbm, idx_v, rows_v, sem):
    wid = jax.lax.axis_index("s") * nc + jax.lax.axis_index("c")
    base = wid * bpw
    pltpu.sync_copy(idx_hbm.at[pl.ds(base, bpw)], idx_v)
    pltpu.async_copy(table_hbm.at[idx_v], rows_v, sem).wait()
    pltpu.sync_copy(rows_v, out_hbm.at[pl.ds(base, bpw)])


def _sc_gather(idxes, table):
    info = plsc.get_sparse_core_info()
    nc, ns = info.num_cores, info.num_subcores
    bpw = _N_TOKENS // (nc * ns)
    call = pl.kernel(
        functools.partial(_sc_gather_body, nc, bpw),
        out_type=jax.ShapeDtypeStruct((_N_TOKENS, _FEATURES_PAD), jnp.float32),
        mesh=plsc.VectorSubcoreMesh(core_axis_name="c", subcore_axis_name="s"),
        scratch_types=[
            pltpu.VMEM((bpw,), jnp.int32),
            pltpu.VMEM((bpw, _FEATURES_PAD), jnp.float32),
            pltpu.SemaphoreType.DMA,
        ],
    )
    return call(idxes, table)


def kernel(z, codebook, W_in, b_in, W_out, b_out):
    idx3, loss11, table = _tc_call(
        z, codebook, W_in, b_in.reshape(1, -1), W_out, b_out.reshape(1, -1))
    idxes = idx3.reshape(_N_TOKENS)
    out = _sc_gather(idxes, table)
    return (out[:, :_FEATURES], loss11.reshape(()), idxes)


# speedup vs baseline: 1.4967x; 1.0093x over previous
import functools

import jax
import jax.numpy as jnp
from jax.experimental import pallas as pl
from jax.experimental.pallas import tpu as pltpu
from jax.experimental.pallas import tpu_sc as plsc

_FEATURES = 192
_FEATURES_PAD = 256
_CODE_FEATURES = 32
_PAGES = 1024
_N_TOKENS = 2048
_BETA = 0.25
_EPS = 1e-12
_TOK_BLK = 2048
_NBLK = _N_TOKENS // _TOK_BLK


def _tc_body(z_ref, cb_ref, w_in_ref, b_in_ref, w_out_ref,
             b_out_ref, idx_ref, loss_ref, p_ref, b_s):
    step = pl.program_id(0)

    @pl.when(step == 0)
    def _prep():
        cb = cb_ref[...]
        cn = cb * jax.lax.rsqrt(
            jnp.mean(cb * cb, axis=1, keepdims=True) + _EPS)
        cnT = cn.T
        c2 = jnp.sum(cnT * cnT, axis=0, keepdims=True)
        b_s[...] = jnp.concatenate([cnT * -2.0, c2], axis=0)
        p = jnp.dot(cn, w_out_ref[...],
                    preferred_element_type=jnp.float32) + b_out_ref[...]
        p_ref[...] = jnp.pad(p, ((0, 0), (0, _FEATURES_PAD - _FEATURES)))

    z = z_ref[...]
    zp = jnp.dot(z, w_in_ref[...],
                 preferred_element_type=jnp.float32) + b_in_ref[...]
    zn = zp * jax.lax.rsqrt(
        jnp.mean(zp * zp, axis=1, keepdims=True) + _EPS)

    zn_aug = jnp.concatenate(
        [zn, jnp.ones((_TOK_BLK, 1), jnp.float32)], axis=1)
    dist = jnp.dot(zn_aug, b_s[...], preferred_element_type=jnp.float32,
                   precision=jax.lax.Precision.HIGHEST)
    m = jnp.min(dist, axis=1, keepdims=True)
    lane = jax.lax.broadcasted_iota(jnp.int32, dist.shape, 1)
    idx = jnp.min(jnp.where(dist == m, lane, _PAGES), axis=1)
    idx_ref[0, 0, :] = idx

    @pl.when(step == 0)
    def _init():
        loss_ref[...] = jnp.zeros_like(loss_ref)

    partial = jnp.sum(zn * zn) + jnp.sum(m)
    loss_ref[...] += jnp.full((1, 1), partial, jnp.float32)

    @pl.when(step == _NBLK - 1)
    def _finish():
        loss_ref[...] *= (1.0 + _BETA) / (_N_TOKENS * _CODE_FEATURES)


_tc_call = pl.pallas_call(
    _tc_body,
    grid=(_NBLK,),
    in_specs=[
        pl.BlockSpec((_TOK_BLK, _FEATURES), lambda i: (i, 0)),
        pl.BlockSpec((_PAGES, _CODE_FEATURES), lambda i: (0, 0)),
        pl.BlockSpec((_FEATURES, _CODE_FEATURES), lambda i: (0, 0)),
        pl.BlockSpec((1, _CODE_FEATURES), lambda i: (0, 0)),
        pl.BlockSpec((_CODE_FEATURES, _FEATURES), lambda i: (0, 0)),
        pl.BlockSpec((1, _FEATURES), lambda i: (0, 0)),
    ],
    out_specs=[
        pl.BlockSpec((1, 1, _TOK_BLK), lambda i: (i, 0, 0)),
        pl.BlockSpec((1, 1), lambda i: (0, 0)),
        pl.BlockSpec((_PAGES, _FEATURES_PAD), lambda i: (0, 0)),
    ],
    out_shape=[
        jax.ShapeDtypeStruct((_NBLK, 1, _TOK_BLK), jnp.int32),
        jax.ShapeDtypeStruct((1, 1), jnp.float32),
        jax.ShapeDtypeStruct((_PAGES, _FEATURES_PAD), jnp.float32),
    ],
    scratch_shapes=[
        pltpu.VMEM((_CODE_FEATURES + 1, _PAGES), jnp.float32),
    ],
)


def _sc_gather_body(nc, bpw, idx_hbm, table_hbm, out_hbm, idx_v, rows_v, sem):
    wid = jax.lax.axis_index("s") * nc + jax.lax.axis_index("c")
    base = wid * bpw
    pltpu.sync_copy(idx_hbm.at[pl.ds(base, bpw)], idx_v)
    pltpu.async_copy(table_hbm.at[idx_v], rows_v, sem).wait()
    pltpu.sync_copy(rows_v, out_hbm.at[pl.ds(base, bpw)])


def _sc_gather(idxes, table):
    info = plsc.get_sparse_core_info()
    nc, ns = info.num_cores, info.num_subcores
    bpw = _N_TOKENS // (nc * ns)
    call = pl.kernel(
        functools.partial(_sc_gather_body, nc, bpw),
        out_type=jax.ShapeDtypeStruct((_N_TOKENS, _FEATURES_PAD), jnp.float32),
        mesh=plsc.VectorSubcoreMesh(core_axis_name="c", subcore_axis_name="s"),
        scratch_types=[
            pltpu.VMEM((bpw,), jnp.int32),
            pltpu.VMEM((bpw, _FEATURES_PAD), jnp.float32),
            pltpu.SemaphoreType.DMA,
        ],
    )
    return call(idxes, table)


def kernel(z, codebook, W_in, b_in, W_out, b_out):
    idx3, loss11, table = _tc_call(
        z, codebook, W_in, b_in.reshape(1, -1), W_out, b_out.reshape(1, -1))
    idxes = idx3.reshape(_N_TOKENS)
    out = _sc_gather(idxes, table)
    return (out[:, :_FEATURES], loss11.reshape(()), idxes)
